# baseline (speedup 1.0000x reference)
import functools

import jax
import jax.numpy as jnp
from jax import lax
from jax.experimental import pallas as pl
from jax.experimental.pallas import tpu as pltpu
from jax.experimental.pallas import tpu_sc as plsc

NC, NS, L = 2, 16, 16
NW = NC * NS

BATCH, FIELDS, D = 4096, 26, 32
VOC = 1000000
FD = FIELDS * D
BRPW = BATCH // NW
NPH = 2
FPH = FIELDS // NPH
SR = FPH * D
KG = BRPW // L


def _rsqrt(v):
    i = lax.bitcast_convert_type(v, jnp.int32)
    y = lax.bitcast_convert_type(jnp.int32(0x5F3759DF) - (i >> 1), jnp.float32)
    for _ in range(3):
        y = y * (1.5 - 0.5 * v * y * y)
    return y


_mesh = plsc.VectorSubcoreMesh(core_axis_name="c", subcore_axis_name="s")


@functools.partial(
    pl.kernel,
    out_type=jax.ShapeDtypeStruct((FD, BATCH), jnp.float32),
    mesh=_mesh,
    compiler_params=pltpu.CompilerParams(use_tc_tiling_on_sc=False),
    scratch_types=[
        pltpu.VMEM((FIELDS, BRPW), jnp.int32),
        pltpu.VMEM((2, D, BRPW), jnp.int32),
        pltpu.VMEM((2, D, BRPW), jnp.float32),
        pltpu.VMEM((SR, BRPW), jnp.float32),
        pltpu.VMEM((D,), jnp.float32),
        pltpu.VMEM((D,), jnp.float32),
        pltpu.SemaphoreType.DMA,
    ],
)
def _embed_ln(xt_hbm, table_hbm, gamma_hbm, beta_hbm, out_hbm,
              idx_v, eidx_v, g_v, slab_v, gamma_v, beta_v, gsem):
    wid = lax.axis_index("s") * NC + lax.axis_index("c")
    col0 = wid * BRPW

    pltpu.sync_copy(xt_hbm.at[:, pl.ds(col0, BRPW)], idx_v)
    pltpu.sync_copy(gamma_hbm, gamma_v)
    pltpu.sync_copy(beta_hbm, beta_v)

    g_lo = gamma_v[pl.ds(0, L)]
    g_hi = gamma_v[pl.ds(L, L)]
    b_lo = beta_v[pl.ds(0, L)]
    b_hi = beta_v[pl.ds(L, L)]

    def build_and_fire(f, par):
        for k in range(KG):
            iv = idx_v[f, pl.ds(k * L, L)]
            eidx_v[par, 0, pl.ds(k * L, L)] = iv
        for d in range(D):
            pltpu.async_copy(
                table_hbm.at[d].at[eidx_v.at[par, 0]], g_v.at[par, d], gsem)

    def wait_field(par):
        for d in range(D):
            pltpu.make_async_copy(
                table_hbm.at[d].at[eidx_v.at[par, 0]], g_v.at[par, d], gsem).wait()

    build_and_fire(0, 0)

    def field_body(f, _):
        par = f & 1
        fl = jnp.where(f >= FPH, f - FPH, f)

        @pl.when(f + 1 < FIELDS)
        def _():
            build_and_fire(f + 1, par ^ 1)

        wait_field(par)

        for k in range(KG):
            s = jnp.zeros((L,), jnp.float32)
            ss = jnp.zeros((L,), jnp.float32)
            for d in range(D):
                g = g_v[par, d, pl.ds(k * L, L)]
                s = s + g
                ss = ss + g * g
            mean = s * (1.0 / D)
            var = ss * (1.0 / D) - mean * mean
            rstd = _rsqrt(var + 1e-5)
            for d in range(D):
                g = g_v[par, d, pl.ds(k * L, L)]
                gam = g_lo[d] if d < L else g_hi[d - L]
                bet = b_lo[d] if d < L else b_hi[d - L]
                slab_v[fl * D + d, pl.ds(k * L, L)] = (g - mean) * rstd * gam + bet

        @pl.when(jnp.logical_or(f == FPH - 1, f == FIELDS - 1))
        def _():
            row0 = pl.multiple_of(jnp.where(f >= FPH, SR, 0), 8)
            pltpu.sync_copy(
                slab_v, out_hbm.at[pl.ds(row0, SR), pl.ds(col0, BRPW)])
        return 0

    lax.fori_loop(0, FIELDS, field_body, 0)


def kernel(x, table, gamma, beta):
    xt = x.T
    tt = table.T
    out_t = _embed_ln(xt, tt, gamma, beta)
    return out_t.T.reshape(BATCH, FD)


# speedup vs baseline: 4.8561x; 4.8561x over previous
import functools

import jax
import jax.numpy as jnp
from jax import lax
from jax.experimental import pallas as pl
from jax.experimental.pallas import tpu as pltpu
from jax.experimental.pallas import tpu_sc as plsc

NC, NS, L = 2, 16, 16
NW = NC * NS

BATCH, FIELDS, D = 4096, 26, 32
VOC = 1000000
R = BATCH * FIELDS
RPW = R // NW
BRPW = BATCH // NW
OBB = 2
CH = OBB * FIELDS
NCH = BRPW // OBB
NGF = CH // L


def _rsqrt(v):
    i = lax.bitcast_convert_type(v, jnp.int32)
    y = lax.bitcast_convert_type(jnp.int32(0x5F3759DF) - (i >> 1), jnp.float32)
    for _ in range(3):
        y = y * (1.5 - 0.5 * v * y * y)
    return y


_mesh = plsc.VectorSubcoreMesh(core_axis_name="c", subcore_axis_name="s")


@functools.partial(
    pl.kernel,
    out_type=jax.ShapeDtypeStruct((BATCH, FIELDS * D), jnp.float32),
    mesh=_mesh,
    scratch_types=[
        pltpu.VMEM((1, RPW), jnp.int32),
        pltpu.VMEM((2 * CH, 8, D), jnp.float32),
        pltpu.VMEM((8, FIELDS * D), jnp.float32),
        pltpu.VMEM((D,), jnp.float32),
        pltpu.VMEM((D,), jnp.float32),
        pltpu.SemaphoreType.DMA,
        pltpu.SemaphoreType.DMA,
    ],
)
def _embed_ln(x_hbm, table_hbm, gamma_hbm, beta_hbm, out_hbm,
              idx_v, slab_v, outbuf, gamma_v, beta_v, gsem0, gsem1):
    wid = lax.axis_index("s") * NC + lax.axis_index("c")

    pltpu.sync_copy(x_hbm.at[wid], idx_v)
    pltpu.sync_copy(gamma_hbm, gamma_v)
    pltpu.sync_copy(beta_hbm, beta_v)

    g_lo = gamma_v[pl.ds(0, L)]
    g_hi = gamma_v[pl.ds(L, L)]
    b_lo = beta_v[pl.ds(0, L)]
    b_hi = beta_v[pl.ds(L, L)]

    lane = lax.iota(jnp.int32, L)
    perms = [lane ^ (1 << k) for k in range(4)]
    _dnums = lax.GatherDimensionNumbers(
        offset_dims=(), collapsed_slice_dims=(0,), start_index_map=(0,))

    def lane_perm(v, p):
        return lax.gather(v, p[:, None], _dnums, (1,),
                          mode=lax.GatherScatterMode.PROMISE_IN_BOUNDS)

    def allreduce_sum(v):
        for p in perms:
            v = v + lane_perm(v, p)
        return v

    def fire_chunk(c, buf, sem):
        def fire_full(g, _):
            iv = idx_v[0, pl.ds(c * CH + g * L, L)]
            for j in range(L):
                q8 = pl.multiple_of((iv[j] >> 3) * 8, 8)
                pltpu.make_async_copy(
                    table_hbm.at[pl.ds(q8, 8)],
                    slab_v.at[buf * CH + g * L + j], sem).start()
            return 0

        lax.fori_loop(0, NGF, fire_full, 0)
        ivt = idx_v[0, pl.ds(c * CH + CH - L, L)]
        for j in range(12, L):
            q8 = pl.multiple_of((ivt[j] >> 3) * 8, 8)
            pltpu.make_async_copy(
                table_hbm.at[pl.ds(q8, 8)],
                slab_v.at[buf * CH + CH - L + j], sem).start()

    def drain_chunk(buf, sem):
        def drain16(g, _):
            for j in range(L):
                pltpu.make_async_copy(
                    table_hbm.at[pl.ds(0, 8)],
                    slab_v.at[buf * CH + g * L + j], sem).wait()
            return 0

        lax.fori_loop(0, NGF, drain16, 0)
        for j in range(12, L):
            pltpu.make_async_copy(
                table_hbm.at[pl.ds(0, 8)],
                slab_v.at[buf * CH + CH - L + j], sem).wait()

    fire_chunk(0, 0, gsem0)

    def chunk_body(c, _):
        par = c & 1

        @pl.when(jnp.logical_and(c + 1 < NCH, par == 0))
        def _():
            fire_chunk(c + 1, 1, gsem1)

        @pl.when(jnp.logical_and(c + 1 < NCH, par == 1))
        def _():
            fire_chunk(c + 1, 0, gsem0)

        @pl.when(par == 0)
        def _():
            drain_chunk(0, gsem0)

        @pl.when(par == 1)
        def _():
            drain_chunk(1, gsem1)

        def ln_rows(ob, _):
            base = c * CH + ob * FIELDS
            iva = idx_v[0, pl.ds(base, L)]
            ivb = idx_v[0, pl.ds(base + 10, L)]
            for f in range(FIELDS):
                slot = ob * FIELDS + f
                sub = (iva[f] if f < L else ivb[f - 10]) & 7
                a = slab_v[par * CH + slot, sub, pl.ds(0, L)]
                bb = slab_v[par * CH + slot, sub, pl.ds(L, L)]
                total = allreduce_sum(a + bb)
                total2 = allreduce_sum(a * a + bb * bb)
                mean = total * (1.0 / D)
                var = total2 * (1.0 / D) - mean * mean
                rstd = _rsqrt(var + 1e-5)
                orow = (c & 3) * OBB + ob
                outbuf[orow, pl.ds(f * D, L)] = (a - mean) * rstd * g_lo + b_lo
                outbuf[orow, pl.ds(f * D + L, L)] = (bb - mean) * rstd * g_hi + b_hi
            return 0

        lax.fori_loop(0, OBB, ln_rows, 0)

        @pl.when((c & 3) == 3)
        def _():
            row0 = pl.multiple_of(wid * BRPW + (c - 3) * OBB, 8)
            pltpu.sync_copy(outbuf, out_hbm.at[pl.ds(row0, 8)])
        return 0

    lax.fori_loop(0, NCH, chunk_body, 0)


def kernel(x, table, gamma, beta):
    x3d = x.reshape(NW, 1, RPW)
    return _embed_ln(x3d, table, gamma, beta)


# speedup vs baseline: 4.8910x; 1.0072x over previous
import functools

import jax
import jax.numpy as jnp
from jax import lax
from jax.experimental import pallas as pl
from jax.experimental.pallas import tpu as pltpu
from jax.experimental.pallas import tpu_sc as plsc

NC, NS, L = 2, 16, 16
NW = NC * NS

BATCH, FIELDS, D = 4096, 26, 32
VOC = 1000000
R = BATCH * FIELDS
RPW = R // NW
BRPW = BATCH // NW
OBB = 2
CH = OBB * FIELDS
NCH = BRPW // OBB
NGF = CH // L


def _rsqrt(v):
    i = lax.bitcast_convert_type(v, jnp.int32)
    y = lax.bitcast_convert_type(jnp.int32(0x5F3759DF) - (i >> 1), jnp.float32)
    for _ in range(3):
        y = y * (1.5 - 0.5 * v * y * y)
    return y


_mesh = plsc.VectorSubcoreMesh(core_axis_name="c", subcore_axis_name="s")


@functools.partial(
    pl.kernel,
    out_type=jax.ShapeDtypeStruct((BATCH, FIELDS * D), jnp.float32),
    mesh=_mesh,
    scratch_types=[
        pltpu.VMEM((1, RPW), jnp.int32),
        pltpu.VMEM((2 * CH, 8, D), jnp.float32),
        pltpu.VMEM((8, FIELDS * D), jnp.float32),
        pltpu.VMEM((D,), jnp.float32),
        pltpu.VMEM((D,), jnp.float32),
        pltpu.SemaphoreType.DMA,
        pltpu.SemaphoreType.DMA,
        pltpu.SemaphoreType.DMA,
        pltpu.SemaphoreType.DMA,
    ],
)
def _embed_ln(x_hbm, table_hbm, gamma_hbm, beta_hbm, out_hbm,
              idx_v, slab_v, outbuf, gamma_v, beta_v, gsem0a, gsem0b, gsem1a, gsem1b):
    wid = lax.axis_index("s") * NC + lax.axis_index("c")

    pltpu.sync_copy(x_hbm.at[wid], idx_v)
    pltpu.sync_copy(gamma_hbm, gamma_v)
    pltpu.sync_copy(beta_hbm, beta_v)

    g_lo = gamma_v[pl.ds(0, L)]
    g_hi = gamma_v[pl.ds(L, L)]
    b_lo = beta_v[pl.ds(0, L)]
    b_hi = beta_v[pl.ds(L, L)]

    lane = lax.iota(jnp.int32, L)
    perms = [lane ^ (1 << k) for k in range(4)]
    _dnums = lax.GatherDimensionNumbers(
        offset_dims=(), collapsed_slice_dims=(0,), start_index_map=(0,))

    def lane_perm(v, p):
        return lax.gather(v, p[:, None], _dnums, (1,),
                          mode=lax.GatherScatterMode.PROMISE_IN_BOUNDS)

    def allreduce_sum(v):
        for p in perms:
            v = v + lane_perm(v, p)
        return v

    def fire_chunk(c, buf, sema, semb):
        def fire_full(g, _):
            iv = idx_v[0, pl.ds(c * CH + g * L, L)]
            for j in range(L):
                q8 = pl.multiple_of((iv[j] >> 3) * 8, 8)
                pltpu.make_async_copy(
                    table_hbm.at[pl.ds(q8, 8)],
                    slab_v.at[buf * CH + g * L + j],
                    sema if j % 2 == 0 else semb).start()
            return 0

        lax.fori_loop(0, NGF, fire_full, 0)
        ivt = idx_v[0, pl.ds(c * CH + CH - L, L)]
        for j in range(12, L):
            q8 = pl.multiple_of((ivt[j] >> 3) * 8, 8)
            pltpu.make_async_copy(
                table_hbm.at[pl.ds(q8, 8)],
                slab_v.at[buf * CH + CH - L + j],
                sema if j % 2 == 0 else semb).start()

    def drain_chunk(buf, sema, semb):
        def drain16(g, _):
            for j in range(L):
                pltpu.make_async_copy(
                    table_hbm.at[pl.ds(0, 8)],
                    slab_v.at[buf * CH + g * L + j],
                    sema if j % 2 == 0 else semb).wait()
            return 0

        lax.fori_loop(0, NGF, drain16, 0)
        for j in range(12, L):
            pltpu.make_async_copy(
                table_hbm.at[pl.ds(0, 8)],
                slab_v.at[buf * CH + CH - L + j],
                sema if j % 2 == 0 else semb).wait()

    fire_chunk(0, 0, gsem0a, gsem0b)

    def chunk_body(c, _):
        par = c & 1

        @pl.when(jnp.logical_and(c + 1 < NCH, par == 0))
        def _():
            fire_chunk(c + 1, 1, gsem1a, gsem1b)

        @pl.when(jnp.logical_and(c + 1 < NCH, par == 1))
        def _():
            fire_chunk(c + 1, 0, gsem0a, gsem0b)

        @pl.when(par == 0)
        def _():
            drain_chunk(0, gsem0a, gsem0b)

        @pl.when(par == 1)
        def _():
            drain_chunk(1, gsem1a, gsem1b)

        def ln_rows(ob, _):
            base = c * CH + ob * FIELDS
            iva = idx_v[0, pl.ds(base, L)]
            ivb = idx_v[0, pl.ds(base + 10, L)]
            for f in range(FIELDS):
                slot = ob * FIELDS + f
                sub = (iva[f] if f < L else ivb[f - 10]) & 7
                a = slab_v[par * CH + slot, sub, pl.ds(0, L)]
                bb = slab_v[par * CH + slot, sub, pl.ds(L, L)]
                total = allreduce_sum(a + bb)
                total2 = allreduce_sum(a * a + bb * bb)
                mean = total * (1.0 / D)
                var = total2 * (1.0 / D) - mean * mean
                rstd = _rsqrt(var + 1e-5)
                orow = (c & 3) * OBB + ob
                outbuf[orow, pl.ds(f * D, L)] = (a - mean) * rstd * g_lo + b_lo
                outbuf[orow, pl.ds(f * D + L, L)] = (bb - mean) * rstd * g_hi + b_hi
            return 0

        lax.fori_loop(0, OBB, ln_rows, 0)

        @pl.when((c & 3) == 3)
        def _():
            row0 = pl.multiple_of(wid * BRPW + (c - 3) * OBB, 8)
            pltpu.sync_copy(outbuf, out_hbm.at[pl.ds(row0, 8)])
        return 0

    lax.fori_loop(0, NCH, chunk_body, 0)


def kernel(x, table, gamma, beta):
    x3d = x.reshape(NW, 1, RPW)
    return _embed_ln(x3d, table, gamma, beta)
